# baseline (speedup 1.0000x reference)
import functools

import jax
import jax.numpy as jnp
from jax import lax
from jax.experimental import pallas as pl

B = 1024
IN_DIM = 4096
H0 = 1024
H1 = 4096
K = 512
EMB = 512
P = H1 // EMB


def _mm_act_kernel(act, in_bf16, x_ref, w_ref, b_ref, o_ref):
    x = x_ref[...]
    if in_bf16 and x.dtype != jnp.bfloat16:
        x = x.astype(jnp.bfloat16)
    y = jnp.dot(x, w_ref[...], preferred_element_type=jnp.float32)
    y = y + b_ref[...]
    if act == "relu":
        y = jax.nn.relu(y)
    elif act == "tanh":
        y = jnp.tanh(y)
    o_ref[...] = y.astype(o_ref.dtype)


def _mm_act(x, w, b, act, out_dtype=jnp.float32, in_bf16=False, bm=256):
    m, k = x.shape
    n = w.shape[1]
    grid = (m // bm,)
    return pl.pallas_call(
        functools.partial(_mm_act_kernel, act, in_bf16),
        grid=grid,
        in_specs=[
            pl.BlockSpec((bm, k), lambda i: (i, 0)),
            pl.BlockSpec((k, n), lambda i: (0, 0)),
            pl.BlockSpec((1, n), lambda i: (0, 0)),
        ],
        out_specs=pl.BlockSpec((bm, n), lambda i: (i, 0)),
        out_shape=jax.ShapeDtypeStruct((m, n), out_dtype),
    )(x, w, b.reshape(1, n))


def _vq_kernel(h1_ref, w2p_ref, b2p_ref, c_ref, ct_ref, emb_ref):
    ztp = jnp.dot(h1_ref[...], w2p_ref[...], preferred_element_type=jnp.float32)
    ztp = ztp + b2p_ref[...]
    c = c_ref[...]
    c2 = jnp.sum(c * c, axis=0, keepdims=True)
    ctb = ct_ref[...].astype(jnp.bfloat16)
    for p in range(P):
        z = ztp[:, p * EMB:(p + 1) * EMB]
        d = c2 - 2.0 * jnp.dot(z, c, preferred_element_type=jnp.float32)
        mn = jnp.min(d, axis=1, keepdims=True)
        iot = lax.broadcasted_iota(jnp.int32, d.shape, 1)
        idx = jnp.min(jnp.where(d == mn, iot, K), axis=1, keepdims=True)
        oh = (iot == idx).astype(jnp.bfloat16)
        emb_ref[p] = jnp.dot(oh, ctb, preferred_element_type=jnp.float32)


def _vq_quantize(h1, w2p, b2p, codebook, ct, bm=256):
    grid = (B // bm,)
    return pl.pallas_call(
        _vq_kernel,
        grid=grid,
        in_specs=[
            pl.BlockSpec((bm, H0), lambda i: (i, 0)),
            pl.BlockSpec((H0, H1), lambda i: (0, 0)),
            pl.BlockSpec((1, H1), lambda i: (0, 0)),
            pl.BlockSpec((EMB, K), lambda i: (0, 0)),
            pl.BlockSpec((K, EMB), lambda i: (0, 0)),
        ],
        out_specs=pl.BlockSpec((P, bm, EMB), lambda i: (0, i, 0)),
        out_shape=jax.ShapeDtypeStruct((P, B, EMB), jnp.float32),
    )(h1, w2p, b2p.reshape(1, H1), codebook, ct)


def _dec_kernel(e_ref, w3p_ref, b3_ref, o_ref):
    acc = b3_ref[...]
    for p in range(P):
        acc = acc + jnp.dot(e_ref[p].astype(jnp.bfloat16),
                            w3p_ref[p * EMB:(p + 1) * EMB, :],
                            preferred_element_type=jnp.float32)
    o_ref[...] = jax.nn.relu(acc).astype(o_ref.dtype)


def _decode1(embt, w3p, b3, bm=256):
    grid = (B // bm,)
    return pl.pallas_call(
        _dec_kernel,
        grid=grid,
        in_specs=[
            pl.BlockSpec((P, bm, EMB), lambda i: (0, i, 0)),
            pl.BlockSpec((H1, H0), lambda i: (0, 0)),
            pl.BlockSpec((1, H0), lambda i: (0, 0)),
        ],
        out_specs=pl.BlockSpec((bm, H0), lambda i: (i, 0)),
        out_shape=jax.ShapeDtypeStruct((B, H0), jnp.bfloat16),
    )(embt, w3p, b3.reshape(1, H0))


def kernel(x, W1, b1, W2, b2, W3, b3, W4, b4, codebook):
    w2p = W2.reshape(H0, EMB, P).transpose(0, 2, 1).reshape(H0, H1)
    b2p = b2.reshape(EMB, P).transpose(1, 0).reshape(H1)
    w3p = (W3.reshape(EMB, P, H0).transpose(1, 0, 2)
           .reshape(H1, H0).astype(jnp.bfloat16))
    w4b = W4.astype(jnp.bfloat16)
    ct = codebook.transpose(1, 0)

    h1 = _mm_act(x, W1, b1, "relu")
    h2 = _mm_act(h1, W2, b2, "none")
    z_e = h2.reshape(B, EMB, P)

    embt = _vq_quantize(h1, w2p, b2p, codebook, ct)

    h3 = _decode1(embt, w3p, b3)
    recon = _mm_act(h3, w4b, b4, "tanh", in_bf16=True)

    emb = embt.transpose(1, 2, 0)
    return (recon, z_e, emb)


# speedup vs baseline: 1.2362x; 1.2362x over previous
import functools

import jax
import jax.numpy as jnp
from jax import lax
from jax.experimental import pallas as pl

B = 1024
IN_DIM = 4096
H0 = 1024
H1 = 4096
K = 512
EMB = 512
P = H1 // EMB


def _mm_act_kernel(act, in_bf16, x_ref, w_ref, b_ref, o_ref):
    x = x_ref[...]
    if in_bf16 and x.dtype != jnp.bfloat16:
        x = x.astype(jnp.bfloat16)
    y = jnp.dot(x, w_ref[...], preferred_element_type=jnp.float32)
    y = y + b_ref[...]
    if act == "relu":
        y = jax.nn.relu(y)
    elif act == "tanh":
        y = jnp.tanh(y)
    o_ref[...] = y.astype(o_ref.dtype)


def _mm_act(x, w, b, act, out_dtype=jnp.float32, in_bf16=False, bm=512):
    m, k = x.shape
    n = w.shape[1]
    grid = (m // bm,)
    return pl.pallas_call(
        functools.partial(_mm_act_kernel, act, in_bf16),
        grid=grid,
        in_specs=[
            pl.BlockSpec((bm, k), lambda i: (i, 0)),
            pl.BlockSpec((k, n), lambda i: (0, 0)),
            pl.BlockSpec((1, n), lambda i: (0, 0)),
        ],
        out_specs=pl.BlockSpec((bm, n), lambda i: (i, 0)),
        out_shape=jax.ShapeDtypeStruct((m, n), out_dtype),
    )(x, w, b.reshape(1, n))


def _vq_kernel(zt_ref, c_ref, ct_ref, emb_ref):
    c = c_ref[...]
    c2 = jnp.sum(c * c, axis=0, keepdims=True)
    d = c2 - 2.0 * jnp.dot(zt_ref[...], c, preferred_element_type=jnp.float32)
    mn = jnp.min(d, axis=1, keepdims=True)
    iot = lax.broadcasted_iota(jnp.int32, d.shape, 1)
    idx = jnp.min(jnp.where(d == mn, iot, K), axis=1, keepdims=True)
    oh = (iot == idx).astype(jnp.bfloat16)
    emb_ref[...] = jnp.dot(oh, ct_ref[...].astype(jnp.bfloat16),
                           preferred_element_type=jnp.float32)


def _vq_quantize(zt, codebook, ct, bm=1024):
    n = zt.shape[0]
    grid = (n // bm,)
    return pl.pallas_call(
        _vq_kernel,
        grid=grid,
        in_specs=[
            pl.BlockSpec((bm, EMB), lambda i: (i, 0)),
            pl.BlockSpec((EMB, K), lambda i: (0, 0)),
            pl.BlockSpec((K, EMB), lambda i: (0, 0)),
        ],
        out_specs=pl.BlockSpec((bm, EMB), lambda i: (i, 0)),
        out_shape=jax.ShapeDtypeStruct((n, EMB), jnp.float32),
    )(zt, codebook, ct)


def kernel(x, W1, b1, W2, b2, W3, b3, W4, b4, codebook):
    w3p = (W3.reshape(EMB, P, H0).transpose(1, 0, 2)
           .reshape(H1, H0).astype(jnp.bfloat16))
    w4b = W4.astype(jnp.bfloat16)
    ct = codebook.transpose(1, 0)

    h1 = _mm_act(x, W1, b1, "relu")
    h2 = _mm_act(h1, W2, b2, "none")
    z_e = h2.reshape(B, EMB, P)

    zt = h2.reshape(B, EMB, P).transpose(0, 2, 1).reshape(B * P, EMB)
    embt = _vq_quantize(zt, codebook, ct)

    zf = embt.reshape(B, H1)
    h3 = _mm_act(zf, w3p, b3, "relu", out_dtype=jnp.bfloat16, in_bf16=True)
    recon = _mm_act(h3, w4b, b4, "tanh", in_bf16=True)

    emb = embt.reshape(B, P, EMB).transpose(0, 2, 1)
    return (recon, z_e, emb)
